# baseline (speedup 1.0000x reference)
import functools

import jax
import jax.numpy as jnp
from jax.experimental import pallas as pl
from jax.experimental.pallas import tpu as pltpu


def _gat_kernel(src_ref, dst_ref, et_ref, h_ref, ep_ref, w1_ref, w2t_ref,
                w3_ref, osrc_ref, oes_ref, st_scr, *, chunk):
    phase = pl.program_id(0)
    c = pl.program_id(1)
    first = jnp.logical_and(phase == 0, c == 0)
    lane = jax.lax.broadcasted_iota(jnp.int32, (1, 128), 1)

    @pl.when(first)
    def _init():
        p = jnp.dot(h_ref[...], w1_ref[...],
                    preferred_element_type=jnp.float32)
        q = jnp.dot(h_ref[...], w3_ref[...],
                    preferred_element_type=jnp.float32)
        st_scr[...] = jnp.zeros(st_scr.shape, jnp.float32)
        st_scr[:, 0:1] = p
        st_scr[:, 1:2] = q
        st_scr[:, 2:3] = jnp.full((st_scr.shape[0], 1), -jnp.inf, jnp.float32)
        osrc_ref[...] = jnp.zeros(osrc_ref.shape, jnp.float32)
        oes_ref[...] = jnp.zeros(oes_ref.shape, jnp.float32)

    def logit(i):
        s = src_ref[i]
        d = dst_ref[i]
        ev = ep_ref[pl.ds(i, 1), :]
        r = jnp.sum(ev * w2t_ref[...], axis=-1, keepdims=True)
        row_s = st_scr[pl.ds(s, 1), :]
        row_d = st_scr[pl.ds(d, 1), :]
        a = row_s[:, 0:1] + row_d[:, 1:2] + r
        a = jnp.where(a >= 0.0, a, 0.01 * a)
        return a, s, d, ev, row_d

    @pl.when(phase == 0)
    def _pass_max():
        def body(i, _):
            a, s, d, ev, row_d = logit(i)
            newm = jnp.maximum(row_d[:, 2:3], a)
            st_scr[pl.ds(d, 1), :] = jnp.where(lane == 2, newm, row_d)
            return 0
        jax.lax.fori_loop(0, chunk, body, 0)

    @pl.when(phase == 1)
    def _pass_sum():
        def body(i, _):
            a, s, d, ev, row_d = logit(i)
            ex = jnp.exp(a - row_d[:, 2:3])
            newz = row_d[:, 3:4] + ex
            st_scr[pl.ds(d, 1), :] = jnp.where(lane == 3, newz, row_d)
            return 0
        jax.lax.fori_loop(0, chunk, body, 0)

    @pl.when(phase == 2)
    def _pass_scatter():
        def body(i, _):
            a, s, d, ev, row_d = logit(i)
            ex = jnp.exp(a - row_d[:, 2:3])
            alpha = ex / (row_d[:, 3:4] + 1e-16)
            row = d * 4 + et_ref[i]
            hs = h_ref[pl.ds(s, 1), :]
            osrc_ref[pl.ds(row, 1), :] = (osrc_ref[pl.ds(row, 1), :]
                                          + alpha * hs)
            oes_ref[pl.ds(row, 1), :] = (oes_ref[pl.ds(row, 1), :]
                                         + alpha * ev)
            return 0
        jax.lax.fori_loop(0, chunk, body, 0)


def kernel(h, e, edge_index, edge_type_idx, attn_w):
    n, d_node = h.shape
    n_edges, d_edge = e.shape
    n_types = 4

    src = edge_index[0].astype(jnp.int32)
    dst = edge_index[1].astype(jnp.int32)
    et = edge_type_idx.astype(jnp.int32)

    w1 = attn_w[:d_node, :]
    w2t = jnp.concatenate(
        [attn_w[d_node:d_node + d_edge, :].reshape(1, d_edge),
         jnp.zeros((1, 128 - d_edge), jnp.float32)], axis=1)
    w3 = attn_w[d_node + d_edge:, :]
    ep = jnp.concatenate(
        [e, jnp.ones((n_edges, 1), jnp.float32),
         jnp.zeros((n_edges, 128 - d_edge - 1), jnp.float32)], axis=1)

    chunk = n_edges
    for cand in (1024, 512, 256, 128):
        if n_edges % cand == 0:
            chunk = cand
            break
    n_chunks = n_edges // chunk

    smem_spec = pl.BlockSpec((chunk,), lambda p, c: (c,),
                             memory_space=pltpu.SMEM)
    full2 = lambda shape: pl.BlockSpec(shape, lambda p, c: (0, 0))

    osrc, oes = pl.pallas_call(
        functools.partial(_gat_kernel, chunk=chunk),
        grid=(3, n_chunks),
        in_specs=[
            smem_spec, smem_spec, smem_spec,
            full2((n, d_node)),
            pl.BlockSpec((chunk, 128), lambda p, c: (c, 0)),
            full2((d_node, 1)), full2((1, 128)), full2((d_node, 1)),
        ],
        out_specs=[
            full2((n * n_types, d_node)),
            full2((n * n_types, 128)),
        ],
        out_shape=[
            jax.ShapeDtypeStruct((n * n_types, d_node), jnp.float32),
            jax.ShapeDtypeStruct((n * n_types, 128), jnp.float32),
        ],
        scratch_shapes=[
            pltpu.VMEM((n, 128), jnp.float32),
        ],
    )(src, dst, et, h, ep, w1, w2t, w3)

    osrc = osrc.reshape(n, n_types, d_node)
    oes = oes.reshape(n, n_types, 128)
    oe = oes[:, :, :d_edge]
    s_sum = oes[:, :, d_edge]
    odst = s_sum[:, :, None] * h[:, None, :]
    h_out = jnp.concatenate([osrc, oe, odst], axis=-1)
    return (h_out, e)
